# baseline (speedup 1.0000x reference)
import functools

import jax
import jax.numpy as jnp
from jax import lax
from jax.experimental import pallas as pl
from jax.experimental.pallas import tpu as pltpu
from jax.experimental.pallas import tpu_sc as plsc

HIDDEN_DIM = 512
BATCH = 16384

_info = plsc.get_sparse_core_info()
NC, NS = _info.num_cores, _info.num_subcores
NW = NC * NS
B_PER_W = BATCH // NW
CHUNK = 64
N_CHUNKS = B_PER_W // CHUNK


def _mesh_kernel():
    mesh = plsc.VectorSubcoreMesh(core_axis_name="c", subcore_axis_name="s")

    @functools.partial(
        pl.kernel,
        mesh=mesh,
        out_type=jax.ShapeDtypeStruct((BATCH, HIDDEN_DIM), jnp.float32),
        scratch_types=[
            pltpu.VMEM((N_CHUNKS, CHUNK), jnp.int32),
            pltpu.VMEM((CHUNK, HIDDEN_DIM), jnp.float32),
            pltpu.VMEM((CHUNK, HIDDEN_DIM), jnp.float32),
            pltpu.SemaphoreType.DMA,
            pltpu.SemaphoreType.DMA,
        ],
    )
    def body(table_hbm, idx_hbm, out_hbm, idx_v, rows0, rows1, sem0, sem1):
        wid = lax.axis_index("s") * NC + lax.axis_index("c")
        base = wid * B_PER_W
        pltpu.sync_copy(idx_hbm.at[wid], idx_v)
        bufs = (rows0, rows1)
        sems = (sem0, sem1)
        cp0 = pltpu.async_copy(table_hbm.at[idx_v.at[0]], bufs[0], sems[0])
        copies = [cp0]
        for k in range(N_CHUNKS):
            copies[k].wait()
            if k + 1 < N_CHUNKS:
                copies.append(
                    pltpu.async_copy(
                        table_hbm.at[idx_v.at[k + 1]],
                        bufs[(k + 1) % 2],
                        sems[(k + 1) % 2],
                    )
                )
            pltpu.sync_copy(
                bufs[k % 2], out_hbm.at[pl.ds(base + k * CHUNK, CHUNK)]
            )

    return body


_sc_lookup = _mesh_kernel()


@jax.jit
def kernel(domain_ids, embed_weight):
    ids = domain_ids.astype(jnp.int32).reshape(NW, N_CHUNKS, CHUNK)
    return _sc_lookup(embed_weight, ids)


# speedup vs baseline: 10.4150x; 10.4150x over previous
import functools

import jax
import jax.numpy as jnp
from jax import lax
from jax.experimental import pallas as pl
from jax.experimental.pallas import tpu as pltpu
from jax.experimental.pallas import tpu_sc as plsc

HIDDEN_DIM = 512
BATCH = 16384
LANES = 16

_info = plsc.get_sparse_core_info()
NC, NS = _info.num_cores, _info.num_subcores
NW = NC * NS
B_PER_W = BATCH // NW
NGRP = B_PER_W // LANES


def _mesh_kernel():
    mesh = plsc.VectorSubcoreMesh(core_axis_name="c", subcore_axis_name="s")

    @functools.partial(
        pl.kernel,
        mesh=mesh,
        out_type=jax.ShapeDtypeStruct((BATCH, HIDDEN_DIM), jnp.float32),
        scratch_types=[
            pltpu.VMEM((B_PER_W,), jnp.int32),
            pltpu.VMEM((2, HIDDEN_DIM), jnp.float32),
            pltpu.SemaphoreType.DMA,
        ],
    )
    def body(table_hbm, idx_hbm, out_hbm, idx_v, tab_v, sem):
        wid = lax.axis_index("s") * NC + lax.axis_index("c")
        base = wid * B_PER_W
        pltpu.sync_copy(idx_hbm.at[wid], idx_v)
        pltpu.sync_copy(table_hbm, tab_v)

        def grp_body(t, _):
            v = idx_v[pl.ds(t * LANES, LANES)]
            row0 = base + t * LANES
            for r in range(LANES):
                pltpu.async_copy(tab_v.at[v[r]], out_hbm.at[row0 + r], sem)
            return 0

        lax.fori_loop(0, NGRP, grp_body, 0)

        def drain_body(t, _):
            pltpu.make_async_copy(tab_v.at[0], out_hbm.at[base], sem).wait()
            return 0

        lax.fori_loop(0, B_PER_W, drain_body, 0)

    return body


_sc_lookup = _mesh_kernel()


@jax.jit
def kernel(domain_ids, embed_weight):
    ids = domain_ids.astype(jnp.int32).reshape(NW, B_PER_W)
    return _sc_lookup(embed_weight, ids)
